# baseline (speedup 1.0000x reference)
import functools

import jax
import jax.numpy as jnp
from jax import lax
from jax.experimental import pallas as pl
from jax.experimental.pallas import tpu as pltpu
from jax.experimental.pallas import tpu_sc as plsc

B = 16384
D = 32
NUM_CORES = 2
NUM_SUBCORES = 16
NW = NUM_CORES * NUM_SUBCORES
B_PER_W = B // NW
CHUNK = 128
N_CHUNKS = B_PER_W // CHUNK


def _make_gather(table_shape):
    mesh = plsc.VectorSubcoreMesh(core_axis_name="c", subcore_axis_name="s")

    @functools.partial(
        pl.kernel,
        mesh=mesh,
        out_type=jax.ShapeDtypeStruct((B, D), jnp.float32),
        scratch_types=[
            pltpu.VMEM((N_CHUNKS, CHUNK), jnp.int32),
            pltpu.VMEM((B_PER_W, D), jnp.float32),
            pltpu.SemaphoreType.DMA,
        ],
        compiler_params=pltpu.CompilerParams(use_tc_tiling_on_sc=False),
    )
    def k(table_hbm, idx_hbm, out_hbm, idx_v, rows_v, sem):
        wid = lax.axis_index("s") * NUM_CORES + lax.axis_index("c")
        base = wid * B_PER_W
        pltpu.sync_copy(idx_hbm.at[wid], idx_v)
        copies = []
        for j in range(N_CHUNKS):
            copies.append(
                pltpu.async_copy(
                    table_hbm.at[idx_v.at[j]],
                    rows_v.at[pl.ds(j * CHUNK, CHUNK)],
                    sem,
                )
            )
        for c in copies:
            c.wait()
        pltpu.sync_copy(rows_v, out_hbm.at[pl.ds(base, B_PER_W)])

    return k


@jax.jit
def kernel(template_table, eventids):
    idx = eventids.astype(jnp.int32).reshape(NW, N_CHUNKS, CHUNK)
    gather = _make_gather(template_table.shape)
    return gather(template_table, idx)


# speedup vs baseline: 1.2395x; 1.2395x over previous
import functools

import jax
import jax.numpy as jnp
from jax import lax
from jax.experimental import pallas as pl
from jax.experimental.pallas import tpu as pltpu
from jax.experimental.pallas import tpu_sc as plsc

B = 16384
D = 32
V = 100001
VPAD = 100096
NUM_CORES = 2
NUM_SUBCORES = 16
NW = NUM_CORES * NUM_SUBCORES
N_TILES = VPAD // 128
SLAB_TILES = 25
SLAB_COLS = SLAB_TILES * 128
N_BANDS = D // 8
IDX_BLK = 4096
POS_SHIFT = 14


def _make_kernel():
    mesh = plsc.VectorSubcoreMesh(core_axis_name="c", subcore_axis_name="s")

    @functools.partial(
        pl.kernel,
        mesh=mesh,
        out_type=jax.ShapeDtypeStruct((B * D,), jnp.float32),
        scratch_types=[
            pltpu.VMEM((IDX_BLK,), jnp.int32),
            pltpu.VMEM((N_BANDS, 8, SLAB_COLS), jnp.float32),
            pltpu.VMEM((B + 16,), jnp.int32),
            pltpu.VMEM((16, D), jnp.float32),
            pltpu.SemaphoreType.DMA,
            pltpu.SemaphoreType.DMA,
        ],
        compiler_params=pltpu.CompilerParams(
            use_tc_tiling_on_sc=True, needs_layout_passes=False
        ),
    )
    def k(tbl_hbm, idx_hbm, out_hbm, idx_v, slab_v, hits_v, ring_v, sem, osem):
        wid = lax.axis_index("s") * NUM_CORES + lax.axis_index("c")
        small = jnp.int32(25 * 14)
        t0 = jnp.where(wid < 14, 25 * wid, small + 24 * (wid - 14))
        ntc = jnp.where(wid < 14, 25, 24)
        slab_t0 = jnp.minimum(t0, N_TILES - SLAB_TILES)
        slab_c0 = slab_t0 * 128
        sel_a = t0 * 128
        sel_b = (t0 + ntc) * 128

        slab_cps = []
        for band in range(N_BANDS):
            slab_cps.append(
                pltpu.async_copy(
                    tbl_hbm.at[pl.ds(band * 8, 8), pl.ds(slab_c0, SLAB_COLS)],
                    slab_v.at[band],
                    sem,
                )
            )
        for c in slab_cps:
            c.wait()

        lane = lax.iota(jnp.int32, 16)
        sel_a_v = jnp.full((16,), 0, jnp.int32) + sel_a
        sel_b_v = jnp.full((16,), 0, jnp.int32) + sel_b
        c0_v = jnp.full((16,), 0, jnp.int32) + slab_c0

        total = jnp.int32(0)
        for blk in range(B // IDX_BLK):
            pltpu.sync_copy(idx_hbm.at[pl.ds(blk * IDX_BLK, IDX_BLK)], idx_v)

            def sel_body(g, off, blk=blk):
                vec = idx_v[pl.ds(g * 16, 16)]
                m = jnp.logical_and(vec >= sel_a_v, vec < sel_b_v)
                cnt = plsc.all_reduce_population_count(m)
                pos_v = lane + (blk * IDX_BLK + g * 16)
                packed = pos_v + lax.shift_left(vec - c0_v, POS_SHIFT)
                plsc.store_compressed(hits_v.at[pl.ds(off, 16)], packed, mask=m)
                return off + cnt[0]

            total = lax.fori_loop(0, IDX_BLK // 16, sel_body, total)

        first_vec = hits_v[pl.ds(0, 16)]
        first = jnp.full((16,), 0, jnp.int32) + first_vec[0]

        @pl.when(total > 0)
        def _():
            hits_v[pl.ds(total, 16)] = first

        band_idx, sub_idx = [], []
        for h in range(2):
            d = lane + h * 16
            band_idx.append(lax.shift_right_logical(d, 3))
            sub_idx.append(d & 7)
        pos_mask = jnp.full((16,), 0, jnp.int32) + ((1 << POS_SHIFT) - 1)
        n_grp = lax.shift_right_logical(total + 15, 4)

        def ext_body(eg, carry):
            pk = hits_v[pl.ds(eg * 16, 16)]
            pos_v = pk & pos_mask
            col_v = lax.shift_right_logical(pk, POS_SHIFT)
            cps = []
            for e in range(16):
                col = jnp.full((16,), 0, jnp.int32) + col_v[e]
                for h in range(2):
                    ring_v[e, pl.ds(h * 16, 16)] = plsc.load_gather(
                        slab_v, [band_idx[h], sub_idx[h], col]
                    )
                cps.append(
                    pltpu.async_copy(
                        ring_v.at[e],
                        out_hbm.at[pl.ds(pos_v[e] * D, D)],
                        osem,
                    )
                )
            for c in cps:
                c.wait()
            return carry

        lax.fori_loop(0, n_grp, ext_body, jnp.int32(0))

    return k


@jax.jit
def kernel(template_table, eventids):
    idx = eventids.astype(jnp.int32)
    tbl_t = template_table.T
    out1d = _make_kernel()(tbl_t, idx)
    return out1d.reshape(B, D)
